# Initial kernel scaffold; baseline (speedup 1.0000x reference)
import jax
import jax.numpy as jnp
from jax.experimental import pallas as pl


def kernel(src, pos, reference_points, W_off, b_off, W_attn, b_attn, Wv, bv, Wo, bo, W1, b1, W2, b2, g1, be1, g2, be2, spatial_shapes):
    raise NotImplementedError("write your pallas kernel here")


# speedup vs baseline: 23.7603x; 23.7603x over previous
import functools

import jax
import jax.numpy as jnp
import numpy as np
from jax import lax
from jax.experimental import pallas as pl
from jax.experimental.pallas import tpu as pltpu
from jax.experimental.pallas import tpu_sc as plsc

BB = 2
DD = 256
NH = 8
NL = 4
NPT = 4
DH = 32
DFF = 1024
SHP = ((100, 100), (50, 50), (25, 25), (13, 13))
QQ = sum(h * w for h, w in SHP)
TQ = 512
QP = 13824
NQT = QP // TQ
NSEG = NL * NPT * 4
NW = 32
ITEMS = BB * NH * QP
IPW = ITEMS // NW
CH = 16
ROUNDS = IPW // CH
RVAL = BB * NH * QP

_STARTS = []
_s = 0
for _h, _w in SHP:
    _STARTS.append(_s)
    _s += _h * _w

_CW = np.repeat(np.array([w for (_h, w) in SHP], np.float32), NPT)[None, :]
_CHT = np.repeat(np.array([h for (h, _w) in SHP], np.float32), NPT)[None, :]
_CS = np.repeat(np.array(_STARTS, np.int32), NPT)[None, :]
_EXP = np.kron(np.eye(NL, dtype=np.float32), np.ones((1, NPT), np.float32))


def _ln(x, g, b):
    m = x.mean(-1, keepdims=True)
    v = ((x - m) ** 2).mean(-1, keepdims=True)
    return (x - m) / jnp.sqrt(v + 1e-5) * g + b


def _stage_a(src_ref, rpx_ref, rpy_ref, wox_ref, box_ref, woy_ref, boy_ref,
             wa_ref, ba_ref, wv_ref, bv_ref, exp_ref, cw_ref, ch_ref, cs_ref,
             val_ref, idx_ref, w_ref):
    b = pl.program_id(0)
    qt = pl.program_id(1)
    s = src_ref[0]
    offx = s @ wox_ref[...] + box_ref[...]
    offy = s @ woy_ref[...] + boy_ref[...]
    logits = s @ wa_ref[...] + ba_ref[...]
    val = s @ wv_ref[...] + bv_ref[...]

    rpx = rpx_ref[0] @ exp_ref[...]
    rpy = rpy_ref[0] @ exp_ref[...]
    cw = cw_ref[...]
    ch = ch_ref[...]
    cs = cs_ref[...]
    cwi = cw.astype(jnp.int32)

    qmask = (qt * TQ + lax.broadcasted_iota(jnp.int32, (TQ, 1), 0)) < QQ

    for h in range(NH):
        a = jax.nn.softmax(logits[:, h * 16:(h + 1) * 16], axis=-1)
        x = rpx * cw + offx[:, h * 16:(h + 1) * 16] - 0.5
        y = rpy * ch + offy[:, h * 16:(h + 1) * 16] - 0.5
        x0 = jnp.floor(x)
        y0 = jnp.floor(y)
        wx1 = x - x0
        wx0 = 1.0 - wx1
        wy1 = y - y0
        wy0 = 1.0 - wy1
        vx0 = (x0 >= 0) & (x0 <= cw - 1)
        vx1 = (x0 + 1 >= 0) & (x0 + 1 <= cw - 1)
        vy0 = (y0 >= 0) & (y0 <= ch - 1)
        vy1 = (y0 + 1 >= 0) & (y0 + 1 <= ch - 1)
        ax0 = jnp.where(vx0, wx0, 0.0)
        ax1 = jnp.where(vx1, wx1, 0.0)
        ay0 = jnp.where(vy0, wy0, 0.0) * a
        ay1 = jnp.where(vy1, wy1, 0.0) * a
        xi0 = jnp.clip(x0, 0.0, cw - 1).astype(jnp.int32)
        xi1 = jnp.clip(x0 + 1, 0.0, cw - 1).astype(jnp.int32)
        yi0 = jnp.clip(y0, 0.0, ch - 1).astype(jnp.int32)
        yi1 = jnp.clip(y0 + 1, 0.0, ch - 1).astype(jnp.int32)
        base = (b * NH + h) * QP + cs
        r0 = base + yi0 * cwi
        r1 = base + yi1 * cwi
        idx64 = jnp.concatenate([r0 + xi0, r0 + xi1, r1 + xi0, r1 + xi1],
                                axis=1)
        w64 = jnp.concatenate([ay0 * ax0, ay0 * ax1, ay1 * ax0, ay1 * ax1],
                              axis=1)
        idx_ref[0, h] = jnp.where(qmask, idx64, 0)
        w_ref[0, h] = jnp.where(qmask, w64, 0.0)
        val_ref[0, h] = val[:, h * DH:(h + 1) * DH]


_GDN = lax.GatherDimensionNumbers(offset_dims=(), collapsed_slice_dims=(0,),
                                  start_index_map=(0,))


def _bcast_lane(v, zeros16, t):
    return lax.gather(v, (zeros16 + t).reshape(16, 1), _GDN, (1,),
                      mode=lax.GatherScatterMode.PROMISE_IN_BOUNDS)


def _sc_body(val_hbm, idx_hbm, w_hbm, out_hbm, idx_v, w_v, rows_v, out_v, sem):
    cid = lax.axis_index("c")
    sid = lax.axis_index("s")
    wid = sid * 2 + cid
    base_item = wid * IPW
    iota16 = lax.broadcasted_iota(jnp.int32, (16,), 0)
    zeros16 = iota16 - iota16

    def round_body(g, carry):
        it0 = base_item + g * CH
        pltpu.sync_copy(idx_hbm.at[pl.ds(it0 * NSEG, CH * NSEG)], idx_v)
        pltpu.sync_copy(w_hbm.at[pl.ds(it0 * NSEG, CH * NSEG)], w_v)
        descs = []
        for k in range((CH * NSEG) // 128):
            descs.append(pltpu.async_copy(
                val_hbm.at[idx_v.at[pl.ds(k * 128, 128)]],
                rows_v.at[pl.ds(k * 128, 128)], sem))
        for d in descs:
            d.wait()

        def item_body(i, c2):
            r0 = i * NSEG
            acc0 = jnp.zeros((16,), jnp.float32)
            acc1 = jnp.zeros((16,), jnp.float32)
            for jb in range(NSEG // 16):
                wv16 = w_v[pl.ds(r0 + jb * 16, 16)]
                for t in range(16):
                    wb = _bcast_lane(wv16, zeros16, t)
                    row = r0 + jb * 16 + t
                    acc0 = acc0 + wb * rows_v[row, pl.ds(0, 16)]
                    acc1 = acc1 + wb * rows_v[row, pl.ds(16, 16)]
            out_v[pl.ds(i * DH, 16)] = acc0
            out_v[pl.ds(i * DH + 16, 16)] = acc1
            return c2

        lax.fori_loop(0, CH, item_body, 0)
        pltpu.sync_copy(out_v, out_hbm.at[pl.ds(it0 * DH, CH * DH)])
        return carry

    lax.fori_loop(0, ROUNDS, round_body, 0)


@functools.lru_cache(maxsize=1)
def _make_sc_sample():
    return pl.kernel(
        _sc_body,
        out_type=jax.ShapeDtypeStruct((ITEMS * DH,), jnp.float32),
        mesh=plsc.VectorSubcoreMesh(core_axis_name="c", subcore_axis_name="s"),
        scratch_types=[
            pltpu.VMEM((CH * NSEG,), jnp.int32),
            pltpu.VMEM((CH * NSEG,), jnp.float32),
            pltpu.VMEM((CH * NSEG, DH), jnp.float32),
            pltpu.VMEM((CH * DH,), jnp.float32),
            pltpu.SemaphoreType.DMA,
        ],
        compiler_params=pltpu.CompilerParams(use_tc_tiling_on_sc=False),
    )


def _stage_b(smp_ref, src_ref, wo_ref, bo_ref, w1_ref, b1_ref, w2_ref, b2_ref,
             g1_ref, be1_ref, g2_ref, be2_ref, out_ref):
    s_cat = jnp.concatenate([smp_ref[0, h] for h in range(NH)], axis=1)
    x = s_cat @ wo_ref[...] + bo_ref[...] + src_ref[0]
    x = _ln(x, g1_ref[...], be1_ref[...])
    hid = jnp.maximum(x @ w1_ref[...] + b1_ref[...], 0.0)
    y = x + hid @ w2_ref[...] + b2_ref[...]
    out_ref[0] = _ln(y, g2_ref[...], be2_ref[...])


def _full(shape):
    nd = len(shape)
    return pl.BlockSpec(shape, lambda b, q: (0,) * nd)


def kernel(src, pos, reference_points, W_off, b_off, W_attn, b_attn, Wv, bv,
           Wo, bo, W1, b1, W2, b2, g1, be1, g2, be2, spatial_shapes):
    f32 = jnp.float32
    srcp = jnp.pad(src, ((0, 0), (0, QP - QQ), (0, 0)))
    rpx = jnp.pad(reference_points[..., 0], ((0, 0), (0, QP - QQ), (0, 0)))
    rpy = jnp.pad(reference_points[..., 1], ((0, 0), (0, QP - QQ), (0, 0)))
    wor = W_off.reshape(DD, NH, NL, NPT, 2)
    wox = wor[..., 0].reshape(DD, NH * NL * NPT)
    woy = wor[..., 1].reshape(DD, NH * NL * NPT)
    bor = b_off.reshape(NH, NL, NPT, 2)
    box = bor[..., 0].reshape(1, NH * NL * NPT)
    boy = bor[..., 1].reshape(1, NH * NL * NPT)
    ba = b_attn.reshape(1, -1)
    bv2 = bv.reshape(1, -1)

    val, idxg, wg = pl.pallas_call(
        _stage_a,
        grid=(BB, NQT),
        in_specs=[
            pl.BlockSpec((1, TQ, DD), lambda b, q: (b, q, 0)),
            pl.BlockSpec((1, TQ, NL), lambda b, q: (b, q, 0)),
            pl.BlockSpec((1, TQ, NL), lambda b, q: (b, q, 0)),
            _full((DD, NH * NL * NPT)),
            _full((1, NH * NL * NPT)),
            _full((DD, NH * NL * NPT)),
            _full((1, NH * NL * NPT)),
            _full((DD, NH * NL * NPT)),
            _full((1, NH * NL * NPT)),
            _full((DD, DD)),
            _full((1, DD)),
            _full((NL, NL * NPT)),
            _full((1, NL * NPT)),
            _full((1, NL * NPT)),
            _full((1, NL * NPT)),
        ],
        out_specs=[
            pl.BlockSpec((1, NH, TQ, DH), lambda b, q: (b, 0, q, 0)),
            pl.BlockSpec((1, NH, TQ, NSEG), lambda b, q: (b, 0, q, 0)),
            pl.BlockSpec((1, NH, TQ, NSEG), lambda b, q: (b, 0, q, 0)),
        ],
        out_shape=[
            jax.ShapeDtypeStruct((BB, NH, QP, DH), f32),
            jax.ShapeDtypeStruct((BB, NH, QP, NSEG), jnp.int32),
            jax.ShapeDtypeStruct((BB, NH, QP, NSEG), f32),
        ],
    )(srcp, rpx, rpy, wox, box, woy, boy, W_attn, ba, Wv, bv2,
      jnp.asarray(_EXP), jnp.asarray(_CW), jnp.asarray(_CHT), jnp.asarray(_CS))

    smp_flat = _make_sc_sample()(val.reshape(RVAL, DH),
                          idxg.reshape(ITEMS * NSEG),
                          wg.reshape(ITEMS * NSEG))
    smp = smp_flat.reshape(BB, NH, QP, DH)

    out = pl.pallas_call(
        _stage_b,
        grid=(BB, NQT),
        in_specs=[
            pl.BlockSpec((1, NH, TQ, DH), lambda b, q: (b, 0, q, 0)),
            pl.BlockSpec((1, TQ, DD), lambda b, q: (b, q, 0)),
            _full((DD, DD)),
            _full((1, DD)),
            _full((DD, DFF)),
            _full((1, DFF)),
            _full((DFF, DD)),
            _full((1, DD)),
            _full((1, DD)),
            _full((1, DD)),
            _full((1, DD)),
            _full((1, DD)),
        ],
        out_specs=pl.BlockSpec((1, TQ, DD), lambda b, q: (b, q, 0)),
        out_shape=jax.ShapeDtypeStruct((BB, QP, DD), f32),
    )(smp, srcp, Wo, bo.reshape(1, -1), W1, b1.reshape(1, -1),
      W2, b2.reshape(1, -1), g1.reshape(1, -1), be1.reshape(1, -1),
      g2.reshape(1, -1), be2.reshape(1, -1))

    return out[:, :QQ, :]


# speedup vs baseline: 26.5463x; 1.1173x over previous
import functools

import jax
import jax.numpy as jnp
import numpy as np
from jax import lax
from jax.experimental import pallas as pl
from jax.experimental.pallas import tpu as pltpu
from jax.experimental.pallas import tpu_sc as plsc

BB = 2
DD = 256
NH = 8
NL = 4
NPT = 4
DH = 32
DFF = 1024
SHP = ((100, 100), (50, 50), (25, 25), (13, 13))
QQ = sum(h * w for h, w in SHP)
TQ = 512
QP = 13824
NQT = QP // TQ
NSEG = NL * NPT * 4
NW = 32
ITEMS = BB * NH * QP
IPW = ITEMS // NW
CH = 16
ROUNDS = IPW // CH
RVAL = BB * NH * QP

_STARTS = []
_s = 0
for _h, _w in SHP:
    _STARTS.append(_s)
    _s += _h * _w

_CW = np.repeat(np.array([w for (_h, w) in SHP], np.float32), NPT)[None, :]
_CHT = np.repeat(np.array([h for (h, _w) in SHP], np.float32), NPT)[None, :]
_CS = np.repeat(np.array(_STARTS, np.int32), NPT)[None, :]
_EXP = np.kron(np.eye(NL, dtype=np.float32), np.ones((1, NPT), np.float32))


def _ln(x, g, b):
    m = x.mean(-1, keepdims=True)
    v = ((x - m) ** 2).mean(-1, keepdims=True)
    return (x - m) / jnp.sqrt(v + 1e-5) * g + b


def _stage_a(src_ref, rpx_ref, rpy_ref, wox_ref, box_ref, woy_ref, boy_ref,
             wa_ref, ba_ref, wv_ref, bv_ref, exp_ref, cw_ref, ch_ref, cs_ref,
             val_ref, idx_ref, w_ref):
    b = pl.program_id(0)
    qt = pl.program_id(1)
    s = src_ref[0]
    offx = s @ wox_ref[...] + box_ref[...]
    offy = s @ woy_ref[...] + boy_ref[...]
    logits = s @ wa_ref[...] + ba_ref[...]
    val = s @ wv_ref[...] + bv_ref[...]

    rpx = rpx_ref[0] @ exp_ref[...]
    rpy = rpy_ref[0] @ exp_ref[...]
    cw = cw_ref[...]
    ch = ch_ref[...]
    cs = cs_ref[...]
    cwi = cw.astype(jnp.int32)

    qmask = (qt * TQ + lax.broadcasted_iota(jnp.int32, (TQ, 1), 0)) < QQ

    for h in range(NH):
        a = jax.nn.softmax(logits[:, h * 16:(h + 1) * 16], axis=-1)
        x = rpx * cw + offx[:, h * 16:(h + 1) * 16] - 0.5
        y = rpy * ch + offy[:, h * 16:(h + 1) * 16] - 0.5
        x0 = jnp.floor(x)
        y0 = jnp.floor(y)
        wx1 = x - x0
        wx0 = 1.0 - wx1
        wy1 = y - y0
        wy0 = 1.0 - wy1
        vx0 = (x0 >= 0) & (x0 <= cw - 1)
        vx1 = (x0 + 1 >= 0) & (x0 + 1 <= cw - 1)
        vy0 = (y0 >= 0) & (y0 <= ch - 1)
        vy1 = (y0 + 1 >= 0) & (y0 + 1 <= ch - 1)
        ax0 = jnp.where(vx0, wx0, 0.0)
        ax1 = jnp.where(vx1, wx1, 0.0)
        ay0 = jnp.where(vy0, wy0, 0.0) * a
        ay1 = jnp.where(vy1, wy1, 0.0) * a
        xi0 = jnp.clip(x0, 0.0, cw - 1).astype(jnp.int32)
        xi1 = jnp.clip(x0 + 1, 0.0, cw - 1).astype(jnp.int32)
        yi0 = jnp.clip(y0, 0.0, ch - 1).astype(jnp.int32)
        yi1 = jnp.clip(y0 + 1, 0.0, ch - 1).astype(jnp.int32)
        base = (b * NH + h) * QP + cs
        r0 = base + yi0 * cwi
        r1 = base + yi1 * cwi
        idx64 = jnp.concatenate([r0 + xi0, r0 + xi1, r1 + xi0, r1 + xi1],
                                axis=1)
        w64 = jnp.concatenate([ay0 * ax0, ay0 * ax1, ay1 * ax0, ay1 * ax1],
                              axis=1)
        idx_ref[0, h] = jnp.where(qmask, idx64, 0)
        w_ref[0, h] = jnp.where(qmask, w64, 0.0)
        val_ref[0, h] = val[:, h * DH:(h + 1) * DH]


_GDN = lax.GatherDimensionNumbers(offset_dims=(), collapsed_slice_dims=(0,),
                                  start_index_map=(0,))


def _bcast_lane(v, zeros16, t):
    return lax.gather(v, (zeros16 + t).reshape(16, 1), _GDN, (1,),
                      mode=lax.GatherScatterMode.PROMISE_IN_BOUNDS)


def _sc_body(val_hbm, idx_hbm, w_hbm, out_hbm, idx_v, w_v, rows_v, out_v,
             sem_iw0, sem_iw1, sem_iw2, sem_iw3, sem_g0, sem_g1):
    cid = lax.axis_index("c")
    sid = lax.axis_index("s")
    wid = sid * 2 + cid
    base_item = wid * IPW
    iota16 = lax.broadcasted_iota(jnp.int32, (16,), 0)
    zeros16 = iota16 - iota16
    sem_iw = [sem_iw0, sem_iw1, sem_iw2, sem_iw3]
    sem_g = [sem_g0, sem_g1]
    ngath = (CH * NSEG) // 128
    last = ROUNDS - 1

    def clamp(g):
        return jnp.minimum(g, last) if not isinstance(g, int) else min(g, last)

    def fire_iw(g, buf):
        it0 = base_item + clamp(g) * CH
        pltpu.async_copy(idx_hbm.at[pl.ds(it0 * NSEG, CH * NSEG)],
                         idx_v.at[buf], sem_iw[buf])
        pltpu.async_copy(w_hbm.at[pl.ds(it0 * NSEG, CH * NSEG)],
                         w_v.at[buf], sem_iw[buf])

    def wait_iw(buf):
        pltpu.make_async_copy(idx_hbm.at[pl.ds(0, CH * NSEG)],
                              idx_v.at[buf], sem_iw[buf]).wait()
        pltpu.make_async_copy(w_hbm.at[pl.ds(0, CH * NSEG)],
                              w_v.at[buf], sem_iw[buf]).wait()

    def fire_gathers(ib, rb):
        for k in range(ngath):
            pltpu.async_copy(val_hbm.at[idx_v.at[ib, pl.ds(k * 128, 128)]],
                             rows_v.at[rb, pl.ds(k * 128, 128)], sem_g[rb])

    def wait_gathers(ib, rb):
        for k in range(ngath):
            pltpu.make_async_copy(
                val_hbm.at[idx_v.at[ib, pl.ds(k * 128, 128)]],
                rows_v.at[rb, pl.ds(k * 128, 128)], sem_g[rb]).wait()

    def compute(g, rb, wb):
        it0 = base_item + g * CH

        def item_body(i, c2):
            r0 = i * NSEG
            accs = [jnp.zeros((16,), jnp.float32) for _ in range(8)]
            for jb in range(NSEG // 16):
                wv16 = w_v[wb, pl.ds(r0 + jb * 16, 16)]
                for t in range(16):
                    wbc = _bcast_lane(wv16, zeros16, t)
                    row = r0 + jb * 16 + t
                    accs[2 * jb] = accs[2 * jb] + wbc * rows_v[rb, row,
                                                               pl.ds(0, 16)]
                    accs[2 * jb + 1] = accs[2 * jb + 1] + wbc * rows_v[
                        rb, row, pl.ds(16, 16)]
            out_v[pl.ds(i * DH, 16)] = ((accs[0] + accs[2])
                                        + (accs[4] + accs[6]))
            out_v[pl.ds(i * DH + 16, 16)] = ((accs[1] + accs[3])
                                             + (accs[5] + accs[7]))
            return c2

        lax.fori_loop(0, CH, item_body, 0)
        pltpu.sync_copy(out_v, out_hbm.at[pl.ds(it0 * DH, CH * DH)])

    fire_iw(0, 0)
    fire_iw(1, 1)
    fire_iw(2, 2)
    wait_iw(0)
    fire_gathers(0, 0)

    def super_round(k, carry):
        for d in range(4):
            g = 4 * k + d
            wait_iw((d + 1) % 4)
            fire_gathers((d + 1) % 4, (d + 1) % 2)
            wait_gathers(d % 4, d % 2)
            compute(g, d % 2, d % 4)
            fire_iw(g + 3, (d + 3) % 4)
        return carry

    lax.fori_loop(0, ROUNDS // 4, super_round, 0)
    wait_iw(1)
    wait_iw(2)
    wait_gathers(0, 0)


@functools.lru_cache(maxsize=1)
def _make_sc_sample():
    return pl.kernel(
        _sc_body,
        out_type=jax.ShapeDtypeStruct((ITEMS * DH,), jnp.float32),
        mesh=plsc.VectorSubcoreMesh(core_axis_name="c", subcore_axis_name="s"),
        scratch_types=[
            pltpu.VMEM((4, CH * NSEG), jnp.int32),
            pltpu.VMEM((4, CH * NSEG), jnp.float32),
            pltpu.VMEM((2, CH * NSEG, DH), jnp.float32),
            pltpu.VMEM((CH * DH,), jnp.float32),
            pltpu.SemaphoreType.DMA,
            pltpu.SemaphoreType.DMA,
            pltpu.SemaphoreType.DMA,
            pltpu.SemaphoreType.DMA,
            pltpu.SemaphoreType.DMA,
            pltpu.SemaphoreType.DMA,
        ],
        compiler_params=pltpu.CompilerParams(use_tc_tiling_on_sc=False),
    )


def _stage_b(smp_ref, src_ref, wo_ref, bo_ref, w1_ref, b1_ref, w2_ref, b2_ref,
             g1_ref, be1_ref, g2_ref, be2_ref, out_ref):
    s_cat = jnp.concatenate([smp_ref[0, h] for h in range(NH)], axis=1)
    x = s_cat @ wo_ref[...] + bo_ref[...] + src_ref[0]
    x = _ln(x, g1_ref[...], be1_ref[...])
    hid = jnp.maximum(x @ w1_ref[...] + b1_ref[...], 0.0)
    y = x + hid @ w2_ref[...] + b2_ref[...]
    out_ref[0] = _ln(y, g2_ref[...], be2_ref[...])


def _full(shape):
    nd = len(shape)
    return pl.BlockSpec(shape, lambda b, q: (0,) * nd)


def kernel(src, pos, reference_points, W_off, b_off, W_attn, b_attn, Wv, bv,
           Wo, bo, W1, b1, W2, b2, g1, be1, g2, be2, spatial_shapes):
    f32 = jnp.float32
    srcp = jnp.pad(src, ((0, 0), (0, QP - QQ), (0, 0)))
    rpx = jnp.pad(reference_points[..., 0], ((0, 0), (0, QP - QQ), (0, 0)))
    rpy = jnp.pad(reference_points[..., 1], ((0, 0), (0, QP - QQ), (0, 0)))
    wor = W_off.reshape(DD, NH, NL, NPT, 2)
    wox = wor[..., 0].reshape(DD, NH * NL * NPT)
    woy = wor[..., 1].reshape(DD, NH * NL * NPT)
    bor = b_off.reshape(NH, NL, NPT, 2)
    box = bor[..., 0].reshape(1, NH * NL * NPT)
    boy = bor[..., 1].reshape(1, NH * NL * NPT)
    ba = b_attn.reshape(1, -1)
    bv2 = bv.reshape(1, -1)

    val, idxg, wg = pl.pallas_call(
        _stage_a,
        grid=(BB, NQT),
        in_specs=[
            pl.BlockSpec((1, TQ, DD), lambda b, q: (b, q, 0)),
            pl.BlockSpec((1, TQ, NL), lambda b, q: (b, q, 0)),
            pl.BlockSpec((1, TQ, NL), lambda b, q: (b, q, 0)),
            _full((DD, NH * NL * NPT)),
            _full((1, NH * NL * NPT)),
            _full((DD, NH * NL * NPT)),
            _full((1, NH * NL * NPT)),
            _full((DD, NH * NL * NPT)),
            _full((1, NH * NL * NPT)),
            _full((DD, DD)),
            _full((1, DD)),
            _full((NL, NL * NPT)),
            _full((1, NL * NPT)),
            _full((1, NL * NPT)),
            _full((1, NL * NPT)),
        ],
        out_specs=[
            pl.BlockSpec((1, NH, TQ, DH), lambda b, q: (b, 0, q, 0)),
            pl.BlockSpec((1, NH, TQ, NSEG), lambda b, q: (b, 0, q, 0)),
            pl.BlockSpec((1, NH, TQ, NSEG), lambda b, q: (b, 0, q, 0)),
        ],
        out_shape=[
            jax.ShapeDtypeStruct((BB, NH, QP, DH), f32),
            jax.ShapeDtypeStruct((BB, NH, QP, NSEG), jnp.int32),
            jax.ShapeDtypeStruct((BB, NH, QP, NSEG), f32),
        ],
    )(srcp, rpx, rpy, wox, box, woy, boy, W_attn, ba, Wv, bv2,
      jnp.asarray(_EXP), jnp.asarray(_CW), jnp.asarray(_CHT), jnp.asarray(_CS))

    smp_flat = _make_sc_sample()(val.reshape(RVAL, DH),
                          idxg.reshape(ITEMS * NSEG),
                          wg.reshape(ITEMS * NSEG))
    smp = smp_flat.reshape(BB, NH, QP, DH)

    out = pl.pallas_call(
        _stage_b,
        grid=(BB, NQT),
        in_specs=[
            pl.BlockSpec((1, NH, TQ, DH), lambda b, q: (b, 0, q, 0)),
            pl.BlockSpec((1, TQ, DD), lambda b, q: (b, q, 0)),
            _full((DD, DD)),
            _full((1, DD)),
            _full((DD, DFF)),
            _full((1, DFF)),
            _full((DFF, DD)),
            _full((1, DD)),
            _full((1, DD)),
            _full((1, DD)),
            _full((1, DD)),
            _full((1, DD)),
        ],
        out_specs=pl.BlockSpec((1, TQ, DD), lambda b, q: (b, q, 0)),
        out_shape=jax.ShapeDtypeStruct((BB, QP, DD), f32),
    )(smp, srcp, Wo, bo.reshape(1, -1), W1, b1.reshape(1, -1),
      W2, b2.reshape(1, -1), g1.reshape(1, -1), be1.reshape(1, -1),
      g2.reshape(1, -1), be2.reshape(1, -1))

    return out[:, :QQ, :]


# speedup vs baseline: 43.5834x; 1.6418x over previous
import functools

import jax
import jax.numpy as jnp
import numpy as np
from jax import lax
from jax.experimental import pallas as pl
from jax.experimental.pallas import tpu as pltpu
from jax.experimental.pallas import tpu_sc as plsc

BB = 2
DD = 256
NH = 8
NL = 4
NPT = 4
DH = 32
DFF = 1024
SHP = ((100, 100), (50, 50), (25, 25), (13, 13))
QQ = sum(h * w for h, w in SHP)
TQ = 512
QP = 13824
NQT = QP // TQ
NW = 32
ITEMS = BB * NH * QP
IPW = ITEMS // NW
CH = 8
ROUNDS = IPW // CH
RVAL = BB * NH * QP
RHALF = RVAL // 2
START1 = SHP[0][0] * SHP[0][1]
NRES = QQ - START1
NP0 = 2 * NPT
NIR = 3 * 16
NWT = 64

_STARTS = []
_s = 0
for _h, _w in SHP:
    _STARTS.append(_s)
    _s += _h * _w
_LSTART = [0, SHP[1][0] * SHP[1][1], SHP[1][0] * SHP[1][1] + SHP[2][0] * SHP[2][1]]

_CW = np.repeat(np.array([w for (_h, w) in SHP], np.float32), NPT)[None, :]
_CHT = np.repeat(np.array([h for (h, _w) in SHP], np.float32), NPT)[None, :]
_CLS = np.repeat(np.array([0] + _LSTART, np.int32), NPT)[None, :] * DH
_EXP = np.kron(np.eye(NL, dtype=np.float32), np.ones((1, NPT), np.float32))


def _ln(x, g, b):
    m = x.mean(-1, keepdims=True)
    v = ((x - m) ** 2).mean(-1, keepdims=True)
    return (x - m) / jnp.sqrt(v + 1e-5) * g + b


def _stage_a(src_ref, rpx_ref, rpy_ref, wox_ref, box_ref, woy_ref, boy_ref,
             wa_ref, ba_ref, wv_ref, bv_ref, exp_ref, cw_ref, ch_ref, cls_ref,
             val_ref, idx0_ref, idxr_ref, w_ref):
    b = pl.program_id(0)
    qt = pl.program_id(1)
    s = src_ref[0]
    offx = s @ wox_ref[...] + box_ref[...]
    offy = s @ woy_ref[...] + boy_ref[...]
    logits = s @ wa_ref[...] + ba_ref[...]
    val = s @ wv_ref[...] + bv_ref[...]

    rpx = rpx_ref[0] @ exp_ref[...]
    rpy = rpy_ref[0] @ exp_ref[...]
    cw = cw_ref[...]
    ch = ch_ref[...]
    cls = cls_ref[...]
    cwi = cw.astype(jnp.int32)

    qmask = (qt * TQ + lax.broadcasted_iota(jnp.int32, (TQ, 1), 0)) < QQ

    for h in range(NH):
        a = jax.nn.softmax(logits[:, h * 16:(h + 1) * 16], axis=-1)
        x = rpx * cw + offx[:, h * 16:(h + 1) * 16] - 0.5
        y = rpy * ch + offy[:, h * 16:(h + 1) * 16] - 0.5
        x0 = jnp.floor(x)
        y0 = jnp.floor(y)
        wx1 = x - x0
        wx0 = 1.0 - wx1
        wy1 = y - y0
        wy0 = 1.0 - wy1
        vx0 = (x0 >= 0) & (x0 <= cw - 1)
        vx1 = (x0 + 1 >= 0) & (x0 + 1 <= cw - 1)
        vy0 = (y0 >= 0) & (y0 <= ch - 1)
        vy1 = (y0 + 1 >= 0) & (y0 + 1 <= ch - 1)
        ax0 = jnp.where(vx0, wx0, 0.0)
        ax1 = jnp.where(vx1, wx1, 0.0)
        ay0 = jnp.where(vy0, wy0, 0.0) * a
        ay1 = jnp.where(vy1, wy1, 0.0) * a
        wax = jnp.where(x0 <= -1.0, ax1, ax0)
        wbx = jnp.where(x0 >= 0.0, ax1, 0.0)
        xs = jnp.clip(x0, 0.0, cw - 1).astype(jnp.int32)
        xi0 = xs
        xi1 = jnp.clip(x0 + 1, 0.0, cw - 1).astype(jnp.int32)
        yi0 = jnp.clip(y0, 0.0, ch - 1).astype(jnp.int32)
        yi1 = jnp.clip(y0 + 1, 0.0, ch - 1).astype(jnp.int32)

        bbase = (b * NH + h) * QP
        g0 = bbase + yi0[:, 0:4] * cwi[:, 0:4] + xs[:, 0:4]
        g1 = bbase + yi1[:, 0:4] * cwi[:, 0:4] + xs[:, 0:4]
        p0 = (g0 >> 1) + (g0 & 1) * RHALF
        p1 = (g1 >> 1) + (g1 & 1) * RHALF
        idx0 = jnp.concatenate([p0, p1], axis=1)
        wl0 = jnp.concatenate([ay0[:, 0:4] * wax[:, 0:4],
                               ay1[:, 0:4] * wax[:, 0:4],
                               ay0[:, 0:4] * wbx[:, 0:4],
                               ay1[:, 0:4] * wbx[:, 0:4]], axis=1)

        lr00 = (cls + (yi0 * cwi + xi0) * DH)
        lr01 = (cls + (yi0 * cwi + xi1) * DH)
        lr10 = (cls + (yi1 * cwi + xi0) * DH)
        lr11 = (cls + (yi1 * cwi + xi1) * DH)
        w00 = ay0 * ax0
        w01 = ay0 * ax1
        w10 = ay1 * ax0
        w11 = ay1 * ax1
        idxr_parts = []
        w_parts = [wl0]
        for l in range(1, NL):
            sl = slice(l * NPT, (l + 1) * NPT)
            idxr_parts += [lr00[:, sl], lr01[:, sl], lr10[:, sl], lr11[:, sl]]
            w_parts += [w00[:, sl], w01[:, sl], w10[:, sl], w11[:, sl]]
        idxr = jnp.concatenate(idxr_parts, axis=1)
        w64 = jnp.concatenate(w_parts, axis=1)

        idx0_ref[0, h] = jnp.where(qmask, idx0, 0)
        idxr_ref[0, h] = jnp.where(qmask, idxr, 0)
        w_ref[0, h] = jnp.where(qmask, w64, 0.0)
        val_ref[0, h] = val[:, h * DH:(h + 1) * DH]


_GDN = lax.GatherDimensionNumbers(offset_dims=(), collapsed_slice_dims=(0,),
                                  start_index_map=(0,))


def _bcast_lane(v, zeros16, t):
    return lax.gather(v, (zeros16 + t).reshape(16, 1), _GDN, (1,),
                      mode=lax.GatherScatterMode.PROMISE_IN_BOUNDS)


def _sc_body(pair_hbm, vf_hbm, idx0_hbm, idxr_hbm, w_hbm, out_hbm,
             val_res, idx0_v, idxr_v, w_v, rows_v, out_v,
             sem_iw0, sem_iw1, sem_iw2, sem_iw3, sem_g0, sem_g1):
    cid = lax.axis_index("c")
    sid = lax.axis_index("s")
    wid = sid * 2 + cid
    base_item = wid * IPW
    iota16 = lax.broadcasted_iota(jnp.int32, (16,), 0)
    zeros16 = iota16 - iota16
    sem_iw = [sem_iw0, sem_iw1, sem_iw2, sem_iw3]
    sem_g = [sem_g0, sem_g1]
    last = ROUNDS - 1

    row0 = (wid >> 1) * QP
    pltpu.sync_copy(vf_hbm.at[pl.ds((row0 + START1) * DH, NRES * DH)], val_res)

    def clamp(g):
        return jnp.minimum(g, last) if not isinstance(g, int) else min(g, last)

    def fire_iw(g, buf):
        it0 = base_item + clamp(g) * CH
        pltpu.async_copy(idx0_hbm.at[pl.ds(it0 * NP0, CH * NP0)],
                         idx0_v.at[buf], sem_iw[buf])
        pltpu.async_copy(idxr_hbm.at[pl.ds(it0 * NIR, CH * NIR)],
                         idxr_v.at[buf], sem_iw[buf])
        pltpu.async_copy(w_hbm.at[pl.ds(it0 * NWT, CH * NWT)],
                         w_v.at[buf], sem_iw[buf])

    def wait_iw(buf):
        pltpu.make_async_copy(idx0_hbm.at[pl.ds(0, CH * NP0)],
                              idx0_v.at[buf], sem_iw[buf]).wait()
        pltpu.make_async_copy(idxr_hbm.at[pl.ds(0, CH * NIR)],
                              idxr_v.at[buf], sem_iw[buf]).wait()
        pltpu.make_async_copy(w_hbm.at[pl.ds(0, CH * NWT)],
                              w_v.at[buf], sem_iw[buf]).wait()

    def fire_gathers(ib, rb):
        pltpu.async_copy(pair_hbm.at[idx0_v.at[ib]], rows_v.at[rb], sem_g[rb])

    def wait_gathers(ib, rb):
        pltpu.make_async_copy(pair_hbm.at[idx0_v.at[ib]], rows_v.at[rb],
                              sem_g[rb]).wait()

    def compute(g, rb, wb):
        it0 = base_item + g * CH

        def item_body(i, c2):
            accs = [jnp.zeros((16,), jnp.float32) for _ in range(8)]
            wv0 = w_v[wb, pl.ds(i * NWT, 16)]
            for u in range(NP0):
                wa = _bcast_lane(wv0, zeros16, u)
                wbb = _bcast_lane(wv0, zeros16, u + 8)
                row = i * NP0 + u
                a0 = 4 * (u % 2)
                accs[a0] = accs[a0] + wa * rows_v[rb, row, pl.ds(0, 16)]
                accs[a0 + 1] = accs[a0 + 1] + wa * rows_v[rb, row,
                                                          pl.ds(16, 16)]
                accs[a0 + 2] = accs[a0 + 2] + wbb * rows_v[rb, row,
                                                           pl.ds(32, 16)]
                accs[a0 + 3] = accs[a0 + 3] + wbb * rows_v[rb, row,
                                                           pl.ds(48, 16)]
            for l in range(1, NL):
                iv = idxr_v[wb, pl.ds(i * NIR + (l - 1) * 16, 16)]
                wv = w_v[wb, pl.ds(i * NWT + l * 16, 16)]
                for t in range(16):
                    wo = _bcast_lane(iv, zeros16, t)
                    wbb = _bcast_lane(wv, zeros16, t)
                    d0 = plsc.load_gather(val_res, [wo + iota16])
                    d1 = plsc.load_gather(val_res, [wo + (iota16 + 16)])
                    a0 = 2 * (t % 4)
                    accs[a0] = accs[a0] + wbb * d0
                    accs[a0 + 1] = accs[a0 + 1] + wbb * d1
            out_v[pl.ds(i * DH, 16)] = ((accs[0] + accs[2])
                                        + (accs[4] + accs[6]))
            out_v[pl.ds(i * DH + 16, 16)] = ((accs[1] + accs[3])
                                             + (accs[5] + accs[7]))
            return c2

        lax.fori_loop(0, CH, item_body, 0)
        pltpu.sync_copy(out_v, out_hbm.at[pl.ds(it0 * DH, CH * DH)])

    fire_iw(0, 0)
    fire_iw(1, 1)
    fire_iw(2, 2)
    wait_iw(0)
    fire_gathers(0, 0)

    def super_round(k, carry):
        for d in range(4):
            g = 4 * k + d
            wait_iw((d + 1) % 4)
            fire_gathers((d + 1) % 4, (d + 1) % 2)
            wait_gathers(d % 4, d % 2)
            compute(g, d % 2, d % 4)
            fire_iw(g + 3, (d + 3) % 4)
        return carry

    lax.fori_loop(0, ROUNDS // 4, super_round, 0)
    wait_iw(1)
    wait_iw(2)
    wait_gathers(0, 0)


@functools.lru_cache(maxsize=1)
def _make_sc_sample():
    return pl.kernel(
        _sc_body,
        out_type=jax.ShapeDtypeStruct((ITEMS * DH,), jnp.float32),
        mesh=plsc.VectorSubcoreMesh(core_axis_name="c", subcore_axis_name="s"),
        scratch_types=[
            pltpu.VMEM((NRES * DH,), jnp.float32),
            pltpu.VMEM((4, CH * NP0), jnp.int32),
            pltpu.VMEM((4, CH * NIR), jnp.int32),
            pltpu.VMEM((4, CH * NWT), jnp.float32),
            pltpu.VMEM((2, CH * NP0, 2 * DH), jnp.float32),
            pltpu.VMEM((CH * DH,), jnp.float32),
            pltpu.SemaphoreType.DMA,
            pltpu.SemaphoreType.DMA,
            pltpu.SemaphoreType.DMA,
            pltpu.SemaphoreType.DMA,
            pltpu.SemaphoreType.DMA,
            pltpu.SemaphoreType.DMA,
        ],
        compiler_params=pltpu.CompilerParams(use_tc_tiling_on_sc=False,
                                             needs_layout_passes=False),
    )


def _stage_b(smp_ref, src_ref, wo_ref, bo_ref, w1_ref, b1_ref, w2_ref, b2_ref,
             g1_ref, be1_ref, g2_ref, be2_ref, out_ref):
    s_cat = jnp.concatenate([smp_ref[0, h] for h in range(NH)], axis=1)
    x = s_cat @ wo_ref[...] + bo_ref[...] + src_ref[0]
    x = _ln(x, g1_ref[...], be1_ref[...])
    hid = jnp.maximum(x @ w1_ref[...] + b1_ref[...], 0.0)
    y = x + hid @ w2_ref[...] + b2_ref[...]
    out_ref[0] = _ln(y, g2_ref[...], be2_ref[...])


def _full(shape):
    nd = len(shape)
    return pl.BlockSpec(shape, lambda b, q: (0,) * nd)


def kernel(src, pos, reference_points, W_off, b_off, W_attn, b_attn, Wv, bv,
           Wo, bo, W1, b1, W2, b2, g1, be1, g2, be2, spatial_shapes):
    f32 = jnp.float32
    srcp = jnp.pad(src, ((0, 0), (0, QP - QQ), (0, 0)))
    rpx = jnp.pad(reference_points[..., 0], ((0, 0), (0, QP - QQ), (0, 0)))
    rpy = jnp.pad(reference_points[..., 1], ((0, 0), (0, QP - QQ), (0, 0)))
    wor = W_off.reshape(DD, NH, NL, NPT, 2)
    wox = wor[..., 0].reshape(DD, NH * NL * NPT)
    woy = wor[..., 1].reshape(DD, NH * NL * NPT)
    bor = b_off.reshape(NH, NL, NPT, 2)
    box = bor[..., 0].reshape(1, NH * NL * NPT)
    boy = bor[..., 1].reshape(1, NH * NL * NPT)
    ba = b_attn.reshape(1, -1)
    bv2 = bv.reshape(1, -1)

    val, idx0, idxr, wg = pl.pallas_call(
        _stage_a,
        grid=(BB, NQT),
        in_specs=[
            pl.BlockSpec((1, TQ, DD), lambda b, q: (b, q, 0)),
            pl.BlockSpec((1, TQ, NL), lambda b, q: (b, q, 0)),
            pl.BlockSpec((1, TQ, NL), lambda b, q: (b, q, 0)),
            _full((DD, NH * NL * NPT)),
            _full((1, NH * NL * NPT)),
            _full((DD, NH * NL * NPT)),
            _full((1, NH * NL * NPT)),
            _full((DD, NH * NL * NPT)),
            _full((1, NH * NL * NPT)),
            _full((DD, DD)),
            _full((1, DD)),
            _full((NL, NL * NPT)),
            _full((1, NL * NPT)),
            _full((1, NL * NPT)),
            _full((1, NL * NPT)),
        ],
        out_specs=[
            pl.BlockSpec((1, NH, TQ, DH), lambda b, q: (b, 0, q, 0)),
            pl.BlockSpec((1, NH, TQ, NP0), lambda b, q: (b, 0, q, 0)),
            pl.BlockSpec((1, NH, TQ, NIR), lambda b, q: (b, 0, q, 0)),
            pl.BlockSpec((1, NH, TQ, NWT), lambda b, q: (b, 0, q, 0)),
        ],
        out_shape=[
            jax.ShapeDtypeStruct((BB, NH, QP, DH), f32),
            jax.ShapeDtypeStruct((BB, NH, QP, NP0), jnp.int32),
            jax.ShapeDtypeStruct((BB, NH, QP, NIR), jnp.int32),
            jax.ShapeDtypeStruct((BB, NH, QP, NWT), f32),
        ],
    )(srcp, rpx, rpy, wox, box, woy, boy, W_attn, ba, Wv, bv2,
      jnp.asarray(_EXP), jnp.asarray(_CW), jnp.asarray(_CHT),
      jnp.asarray(_CLS))

    vf = val.reshape(RVAL * DH)
    even = vf.reshape(RHALF, 2 * DH)
    odd = jnp.concatenate([vf[DH:], jnp.zeros((DH,), f32)]).reshape(
        RHALF, 2 * DH)
    val_pair = jnp.concatenate([even, odd], axis=0)

    smp_flat = _make_sc_sample()(val_pair, vf,
                                 idx0.reshape(ITEMS * NP0),
                                 idxr.reshape(ITEMS * NIR),
                                 wg.reshape(ITEMS * NWT))
    smp = smp_flat.reshape(BB, NH, QP, DH)

    out = pl.pallas_call(
        _stage_b,
        grid=(BB, NQT),
        in_specs=[
            pl.BlockSpec((1, NH, TQ, DH), lambda b, q: (b, 0, q, 0)),
            pl.BlockSpec((1, TQ, DD), lambda b, q: (b, q, 0)),
            _full((DD, DD)),
            _full((1, DD)),
            _full((DD, DFF)),
            _full((1, DFF)),
            _full((DFF, DD)),
            _full((1, DD)),
            _full((1, DD)),
            _full((1, DD)),
            _full((1, DD)),
            _full((1, DD)),
        ],
        out_specs=pl.BlockSpec((1, TQ, DD), lambda b, q: (b, q, 0)),
        out_shape=jax.ShapeDtypeStruct((BB, QP, DD), f32),
    )(smp, srcp, Wo, bo.reshape(1, -1), W1, b1.reshape(1, -1),
      W2, b2.reshape(1, -1), g1.reshape(1, -1), be1.reshape(1, -1),
      g2.reshape(1, -1), be2.reshape(1, -1))

    return out[:, :QQ, :]


# speedup vs baseline: 57.6403x; 1.3225x over previous
import functools

import jax
import jax.numpy as jnp
import numpy as np
from jax import lax
from jax.experimental import pallas as pl
from jax.experimental.pallas import tpu as pltpu
from jax.experimental.pallas import tpu_sc as plsc

BB = 2
DD = 256
NH = 8
NL = 4
NPT = 4
DH = 32
DFF = 1024
SHP = ((100, 100), (50, 50), (25, 25), (13, 13))
QQ = sum(h * w for h, w in SHP)
TQ = 512
QP = 13824
NQT = QP // TQ
NW = 32
ITEMS = BB * NH * QP
IPW = ITEMS // NW
CH = 8
ROUNDS = IPW // CH
RVAL = BB * NH * QP
RHALF = RVAL // 2
START1 = SHP[0][0] * SHP[0][1]
NRES = QQ - START1
NP0 = 2 * NPT
NIR = 3 * 16
NWT = 64

_STARTS = []
_s = 0
for _h, _w in SHP:
    _STARTS.append(_s)
    _s += _h * _w
_LSTART = [0, SHP[1][0] * SHP[1][1], SHP[1][0] * SHP[1][1] + SHP[2][0] * SHP[2][1]]

_COLS = np.arange(NH * NL * NPT)
_HCOL = _COLS // (NL * NPT)
_LCOL = (_COLS % (NL * NPT)) // NPT
_PCOL = _COLS % NPT
_CW = np.array([SHP[l][1] for l in _LCOL], np.float32)[None, :]
_CHT = np.array([SHP[l][0] for l in _LCOL], np.float32)[None, :]
_CLS = np.array([0 if l == 0 else _LSTART[l - 1] for l in _LCOL],
                np.float32)[None, :]
_CHQ2 = (np.arange(NH * NP0) // NP0 * (QP // 2)).astype(np.float32)[None, :]
_EXP = (np.arange(NL)[:, None] == _LCOL[None, :]).astype(np.float32)
_GSUM = (_COLS[:, None] // 16 == _COLS[None, :] // 16).astype(np.float32)


def _perm(out_cols, n_out, mask=None):
    m = np.zeros((NH * NL * NPT, n_out), np.float32)
    for c in range(NH * NL * NPT):
        if mask is None or mask[c]:
            m[c, out_cols[c]] = 1.0
    return m


_P0 = [_perm(_HCOL * NP0 + cy * NPT + _PCOL, NH * NP0, _LCOL == 0)
       for cy in range(2)]
_PR = [_perm(_HCOL * NIR + (_LCOL - 1) * 16 + cidx * NPT + _PCOL, NH * NIR,
             _LCOL >= 1) for cidx in range(4)]
_PWA = [_perm(_HCOL * NWT + cy * NPT + _PCOL, NH * NWT, _LCOL == 0)
        for cy in range(2)]
_PWB = [_perm(_HCOL * NWT + NP0 + cy * NPT + _PCOL, NH * NWT, _LCOL == 0)
        for cy in range(2)]
_PWR = [_perm(_HCOL * NWT + 16 * _LCOL + cidx * NPT + _PCOL, NH * NWT,
              _LCOL >= 1) for cidx in range(4)]


def _ln(x, g, b):
    m = x.mean(-1, keepdims=True)
    v = ((x - m) ** 2).mean(-1, keepdims=True)
    return (x - m) / jnp.sqrt(v + 1e-5) * g + b


def _hp(a, b):
    return jnp.dot(a, b, precision=lax.Precision.HIGHEST)


def _pex(x, p):
    xh = jnp.floor(x * (1.0 / 256.0)) * 256.0
    return (xh @ p) + ((x - xh) @ p)


def _pw(x, p):
    xh = x.astype(jnp.bfloat16).astype(jnp.float32)
    return (xh @ p) + ((x - xh) @ p)


def _stage_a(src_ref, rpx_ref, rpy_ref, wox_ref, box_ref, woy_ref, boy_ref,
             wa_ref, ba_ref, wv_ref, bv_ref, exp_ref, cw_ref, ch_ref, cls_ref,
             chq_ref, gsum_ref, p0a_ref, p0b_ref, pr0_ref, pr1_ref, pr2_ref,
             pr3_ref, pwa0_ref, pwa1_ref, pwb0_ref, pwb1_ref, pwr0_ref,
             pwr1_ref, pwr2_ref, pwr3_ref, val_ref, idx0_ref, idxr_ref,
             w_ref):
    b = pl.program_id(0)
    qt = pl.program_id(1)
    s = src_ref[0]
    offx = s @ wox_ref[...] + box_ref[...]
    offy = s @ woy_ref[...] + boy_ref[...]
    logits = s @ wa_ref[...] + ba_ref[...]
    val = s @ wv_ref[...] + bv_ref[...]

    rpx = _hp(rpx_ref[0], exp_ref[...])
    rpy = _hp(rpy_ref[0], exp_ref[...])
    cw = cw_ref[...]
    ch = ch_ref[...]

    e = jnp.exp(logits - logits.max(axis=-1, keepdims=True))
    a = e / _pw(e, gsum_ref[...])

    x = rpx * cw + offx - 0.5
    y = rpy * ch + offy - 0.5
    x0 = jnp.floor(x)
    y0 = jnp.floor(y)
    wx1 = x - x0
    wx0 = 1.0 - wx1
    wy1 = y - y0
    wy0 = 1.0 - wy1
    ax0 = jnp.where((x0 >= 0) & (x0 <= cw - 1), wx0, 0.0)
    ax1 = jnp.where((x0 + 1 >= 0) & (x0 + 1 <= cw - 1), wx1, 0.0)
    ay0 = jnp.where((y0 >= 0) & (y0 <= ch - 1), wy0, 0.0) * a
    ay1 = jnp.where((y0 + 1 >= 0) & (y0 + 1 <= ch - 1), wy1, 0.0) * a
    wax = jnp.where(x0 <= -1.0, ax1, ax0)
    wbx = jnp.where(x0 >= 0.0, ax1, 0.0)
    xi0 = jnp.clip(x0, 0.0, cw - 1)
    xi1 = jnp.clip(x0 + 1, 0.0, cw - 1)
    yi0 = jnp.clip(y0, 0.0, ch - 1)
    yi1 = jnp.clip(y0 + 1, 0.0, ch - 1)

    gs0 = yi0 * cw + xi0
    gs1 = yi1 * cw + xi0
    h0 = jnp.floor(gs0 * 0.5)
    h1 = jnp.floor(gs1 * 0.5)
    q0 = gs0 - 2.0 * h0
    q1 = gs1 - 2.0 * h1
    bb = (b * (NH * QP // 2)).astype(jnp.float32)

    cls = cls_ref[...]
    lr00 = cls + yi0 * cw + xi0
    lr01 = cls + yi0 * cw + xi1
    lr10 = cls + yi1 * cw + xi0
    lr11 = cls + yi1 * cw + xi1
    w00 = ay0 * ax0
    w01 = ay0 * ax1
    w10 = ay1 * ax0
    w11 = ay1 * ax1

    idx0_f = (_pex(h0, p0a_ref[...]) + _pex(h1, p0b_ref[...])
              + ((q0 @ p0a_ref[...]) + (q1 @ p0b_ref[...])) * RHALF
              + chq_ref[...] + bb)
    idxr_f = (_pex(lr00, pr0_ref[...]) + _pex(lr01, pr1_ref[...])
              + _pex(lr10, pr2_ref[...]) + _pex(lr11, pr3_ref[...])
              ) * float(DH)
    w_all = (_pw(ay0 * wax, pwa0_ref[...]) + _pw(ay1 * wax, pwa1_ref[...])
             + _pw(ay0 * wbx, pwb0_ref[...]) + _pw(ay1 * wbx, pwb1_ref[...])
             + _pw(w00, pwr0_ref[...]) + _pw(w01, pwr1_ref[...])
             + _pw(w10, pwr2_ref[...]) + _pw(w11, pwr3_ref[...]))

    qmask = (qt * TQ + lax.broadcasted_iota(jnp.int32, (TQ, 1), 0)) < QQ
    idx0_i = jnp.where(qmask, idx0_f + 0.5, 0.0).astype(jnp.int32)
    idxr_i = jnp.where(qmask, idxr_f + 0.5, 0.0).astype(jnp.int32)
    w_all = jnp.where(qmask, w_all, 0.0)

    for h in range(NH):
        idx0_ref[0, h] = idx0_i[:, h * NP0:(h + 1) * NP0]
        idxr_ref[0, h] = idxr_i[:, h * NIR:(h + 1) * NIR]
        w_ref[0, h] = w_all[:, h * NWT:(h + 1) * NWT]
        val_ref[0, h] = val[:, h * DH:(h + 1) * DH]


_GDN = lax.GatherDimensionNumbers(offset_dims=(), collapsed_slice_dims=(0,),
                                  start_index_map=(0,))


def _bcast_lane(v, zeros16, t):
    return lax.gather(v, (zeros16 + t).reshape(16, 1), _GDN, (1,),
                      mode=lax.GatherScatterMode.PROMISE_IN_BOUNDS)


def _sc_body(pair_hbm, vf_hbm, idx0_hbm, idxr_hbm, w_hbm, out_hbm,
             val_res, idx0_v, idxr_v, w_v, rows_v, out_v,
             sem_iw0, sem_iw1, sem_iw2, sem_iw3, sem_g0, sem_g1):
    cid = lax.axis_index("c")
    sid = lax.axis_index("s")
    wid = sid * 2 + cid
    base_item = wid * IPW
    iota16 = lax.broadcasted_iota(jnp.int32, (16,), 0)
    zeros16 = iota16 - iota16
    sem_iw = [sem_iw0, sem_iw1, sem_iw2, sem_iw3]
    sem_g = [sem_g0, sem_g1]
    last = ROUNDS - 1

    row0 = (wid >> 1) * QP
    pltpu.sync_copy(vf_hbm.at[pl.ds((row0 + START1) * DH, NRES * DH)], val_res)

    def clamp(g):
        return jnp.minimum(g, last) if not isinstance(g, int) else min(g, last)

    def fire_iw(g, buf):
        it0 = base_item + clamp(g) * CH
        pltpu.async_copy(idx0_hbm.at[pl.ds(it0 * NP0, CH * NP0)],
                         idx0_v.at[buf], sem_iw[buf])
        pltpu.async_copy(idxr_hbm.at[pl.ds(it0 * NIR, CH * NIR)],
                         idxr_v.at[buf], sem_iw[buf])
        pltpu.async_copy(w_hbm.at[pl.ds(it0 * NWT, CH * NWT)],
                         w_v.at[buf], sem_iw[buf])

    def wait_iw(buf):
        pltpu.make_async_copy(idx0_hbm.at[pl.ds(0, CH * NP0)],
                              idx0_v.at[buf], sem_iw[buf]).wait()
        pltpu.make_async_copy(idxr_hbm.at[pl.ds(0, CH * NIR)],
                              idxr_v.at[buf], sem_iw[buf]).wait()
        pltpu.make_async_copy(w_hbm.at[pl.ds(0, CH * NWT)],
                              w_v.at[buf], sem_iw[buf]).wait()

    def fire_gathers(ib, rb):
        pltpu.async_copy(pair_hbm.at[idx0_v.at[ib]], rows_v.at[rb], sem_g[rb])

    def wait_gathers(ib, rb):
        pltpu.make_async_copy(pair_hbm.at[idx0_v.at[ib]], rows_v.at[rb],
                              sem_g[rb]).wait()

    def compute(g, rb, wb):
        it0 = base_item + g * CH

        def item_body(i, c2):
            accs = [jnp.zeros((16,), jnp.float32) for _ in range(8)]
            wv0 = w_v[wb, pl.ds(i * NWT, 16)]
            for u in range(NP0):
                wa = _bcast_lane(wv0, zeros16, u)
                wbb = _bcast_lane(wv0, zeros16, u + 8)
                row = i * NP0 + u
                a0 = 4 * (u % 2)
                accs[a0] = accs[a0] + wa * rows_v[rb, row, pl.ds(0, 16)]
                accs[a0 + 1] = accs[a0 + 1] + wa * rows_v[rb, row,
                                                          pl.ds(16, 16)]
                accs[a0 + 2] = accs[a0 + 2] + wbb * rows_v[rb, row,
                                                           pl.ds(32, 16)]
                accs[a0 + 3] = accs[a0 + 3] + wbb * rows_v[rb, row,
                                                           pl.ds(48, 16)]
            for l in range(1, NL):
                iv = idxr_v[wb, pl.ds(i * NIR + (l - 1) * 16, 16)]
                wv = w_v[wb, pl.ds(i * NWT + l * 16, 16)]
                for t in range(16):
                    wo = _bcast_lane(iv, zeros16, t)
                    wbb = _bcast_lane(wv, zeros16, t)
                    d0 = plsc.load_gather(val_res, [wo + iota16])
                    d1 = plsc.load_gather(val_res, [wo + (iota16 + 16)])
                    a0 = 2 * (t % 4)
                    accs[a0] = accs[a0] + wbb * d0
                    accs[a0 + 1] = accs[a0 + 1] + wbb * d1
            out_v[pl.ds(i * DH, 16)] = ((accs[0] + accs[2])
                                        + (accs[4] + accs[6]))
            out_v[pl.ds(i * DH + 16, 16)] = ((accs[1] + accs[3])
                                             + (accs[5] + accs[7]))
            return c2

        lax.fori_loop(0, CH, item_body, 0)
        pltpu.sync_copy(out_v, out_hbm.at[pl.ds(it0 * DH, CH * DH)])

    fire_iw(0, 0)
    fire_iw(1, 1)
    fire_iw(2, 2)
    wait_iw(0)
    fire_gathers(0, 0)

    def super_round(k, carry):
        for d in range(4):
            g = 4 * k + d
            wait_iw((d + 1) % 4)
            fire_gathers((d + 1) % 4, (d + 1) % 2)
            wait_gathers(d % 4, d % 2)
            compute(g, d % 2, d % 4)
            fire_iw(g + 3, (d + 3) % 4)
        return carry

    lax.fori_loop(0, ROUNDS // 4, super_round, 0)
    wait_iw(1)
    wait_iw(2)
    wait_gathers(0, 0)


@functools.lru_cache(maxsize=1)
def _make_sc_sample():
    return pl.kernel(
        _sc_body,
        out_type=jax.ShapeDtypeStruct((ITEMS * DH,), jnp.float32),
        mesh=plsc.VectorSubcoreMesh(core_axis_name="c", subcore_axis_name="s"),
        scratch_types=[
            pltpu.VMEM((NRES * DH,), jnp.float32),
            pltpu.VMEM((4, CH * NP0), jnp.int32),
            pltpu.VMEM((4, CH * NIR), jnp.int32),
            pltpu.VMEM((4, CH * NWT), jnp.float32),
            pltpu.VMEM((2, CH * NP0, 2 * DH), jnp.float32),
            pltpu.VMEM((CH * DH,), jnp.float32),
            pltpu.SemaphoreType.DMA,
            pltpu.SemaphoreType.DMA,
            pltpu.SemaphoreType.DMA,
            pltpu.SemaphoreType.DMA,
            pltpu.SemaphoreType.DMA,
            pltpu.SemaphoreType.DMA,
        ],
        compiler_params=pltpu.CompilerParams(use_tc_tiling_on_sc=False,
                                             needs_layout_passes=False),
    )


def _stage_b(smp_ref, src_ref, wo_ref, bo_ref, w1_ref, b1_ref, w2_ref, b2_ref,
             g1_ref, be1_ref, g2_ref, be2_ref, out_ref):
    s_cat = jnp.concatenate([smp_ref[0, h] for h in range(NH)], axis=1)
    x = s_cat @ wo_ref[...] + bo_ref[...] + src_ref[0]
    x = _ln(x, g1_ref[...], be1_ref[...])
    hid = jnp.maximum(x @ w1_ref[...] + b1_ref[...], 0.0)
    y = x + hid @ w2_ref[...] + b2_ref[...]
    out_ref[0] = _ln(y, g2_ref[...], be2_ref[...])


def _full(shape):
    nd = len(shape)
    return pl.BlockSpec(shape, lambda b, q: (0,) * nd)


def kernel(src, pos, reference_points, W_off, b_off, W_attn, b_attn, Wv, bv,
           Wo, bo, W1, b1, W2, b2, g1, be1, g2, be2, spatial_shapes):
    f32 = jnp.float32
    srcp = jnp.pad(src, ((0, 0), (0, QP - QQ), (0, 0)))
    rpx = jnp.pad(reference_points[..., 0], ((0, 0), (0, QP - QQ), (0, 0)))
    rpy = jnp.pad(reference_points[..., 1], ((0, 0), (0, QP - QQ), (0, 0)))
    wor = W_off.reshape(DD, NH, NL, NPT, 2)
    wox = wor[..., 0].reshape(DD, NH * NL * NPT)
    woy = wor[..., 1].reshape(DD, NH * NL * NPT)
    bor = b_off.reshape(NH, NL, NPT, 2)
    box = bor[..., 0].reshape(1, NH * NL * NPT)
    boy = bor[..., 1].reshape(1, NH * NL * NPT)
    ba = b_attn.reshape(1, -1)
    bv2 = bv.reshape(1, -1)

    nc = NH * NL * NPT
    val, idx0, idxr, wg = pl.pallas_call(
        _stage_a,
        grid=(BB, NQT),
        in_specs=[
            pl.BlockSpec((1, TQ, DD), lambda b, q: (b, q, 0)),
            pl.BlockSpec((1, TQ, NL), lambda b, q: (b, q, 0)),
            pl.BlockSpec((1, TQ, NL), lambda b, q: (b, q, 0)),
            _full((DD, nc)),
            _full((1, nc)),
            _full((DD, nc)),
            _full((1, nc)),
            _full((DD, nc)),
            _full((1, nc)),
            _full((DD, DD)),
            _full((1, DD)),
            _full((NL, nc)),
            _full((1, nc)),
            _full((1, nc)),
            _full((1, nc)),
            _full((1, NH * NP0)),
            _full((nc, nc)),
            _full((nc, NH * NP0)),
            _full((nc, NH * NP0)),
            _full((nc, NH * NIR)),
            _full((nc, NH * NIR)),
            _full((nc, NH * NIR)),
            _full((nc, NH * NIR)),
            _full((nc, NH * NWT)),
            _full((nc, NH * NWT)),
            _full((nc, NH * NWT)),
            _full((nc, NH * NWT)),
            _full((nc, NH * NWT)),
            _full((nc, NH * NWT)),
            _full((nc, NH * NWT)),
            _full((nc, NH * NWT)),
        ],
        out_specs=[
            pl.BlockSpec((1, NH, TQ, DH), lambda b, q: (b, 0, q, 0)),
            pl.BlockSpec((1, NH, TQ, NP0), lambda b, q: (b, 0, q, 0)),
            pl.BlockSpec((1, NH, TQ, NIR), lambda b, q: (b, 0, q, 0)),
            pl.BlockSpec((1, NH, TQ, NWT), lambda b, q: (b, 0, q, 0)),
        ],
        out_shape=[
            jax.ShapeDtypeStruct((BB, NH, QP, DH), f32),
            jax.ShapeDtypeStruct((BB, NH, QP, NP0), jnp.int32),
            jax.ShapeDtypeStruct((BB, NH, QP, NIR), jnp.int32),
            jax.ShapeDtypeStruct((BB, NH, QP, NWT), f32),
        ],
    )(srcp, rpx, rpy, wox, box, woy, boy, W_attn, ba, Wv, bv2,
      jnp.asarray(_EXP), jnp.asarray(_CW), jnp.asarray(_CHT),
      jnp.asarray(_CLS), jnp.asarray(_CHQ2), jnp.asarray(_GSUM),
      jnp.asarray(_P0[0]), jnp.asarray(_P0[1]),
      jnp.asarray(_PR[0]), jnp.asarray(_PR[1]),
      jnp.asarray(_PR[2]), jnp.asarray(_PR[3]),
      jnp.asarray(_PWA[0]), jnp.asarray(_PWA[1]),
      jnp.asarray(_PWB[0]), jnp.asarray(_PWB[1]),
      jnp.asarray(_PWR[0]), jnp.asarray(_PWR[1]),
      jnp.asarray(_PWR[2]), jnp.asarray(_PWR[3]))

    vf = val.reshape(RVAL * DH)
    even = vf.reshape(RHALF, 2 * DH)
    odd = jnp.concatenate([vf[DH:], jnp.zeros((DH,), f32)]).reshape(
        RHALF, 2 * DH)
    val_pair = jnp.concatenate([even, odd], axis=0)

    smp_flat = _make_sc_sample()(val_pair, vf,
                                 idx0.reshape(ITEMS * NP0),
                                 idxr.reshape(ITEMS * NIR),
                                 wg.reshape(ITEMS * NWT))
    smp = smp_flat.reshape(BB, NH, QP, DH)

    out = pl.pallas_call(
        _stage_b,
        grid=(BB, NQT),
        in_specs=[
            pl.BlockSpec((1, NH, TQ, DH), lambda b, q: (b, 0, q, 0)),
            pl.BlockSpec((1, TQ, DD), lambda b, q: (b, q, 0)),
            _full((DD, DD)),
            _full((1, DD)),
            _full((DD, DFF)),
            _full((1, DFF)),
            _full((DFF, DD)),
            _full((1, DD)),
            _full((1, DD)),
            _full((1, DD)),
            _full((1, DD)),
            _full((1, DD)),
        ],
        out_specs=pl.BlockSpec((1, TQ, DD), lambda b, q: (b, q, 0)),
        out_shape=jax.ShapeDtypeStruct((BB, QP, DD), f32),
    )(smp, srcp, Wo, bo.reshape(1, -1), W1, b1.reshape(1, -1),
      W2, b2.reshape(1, -1), g1.reshape(1, -1), be1.reshape(1, -1),
      g2.reshape(1, -1), be2.reshape(1, -1))

    return out[:, :QQ, :]


# speedup vs baseline: 58.3521x; 1.0124x over previous
import functools

import jax
import jax.numpy as jnp
import numpy as np
from jax import lax
from jax.experimental import pallas as pl
from jax.experimental.pallas import tpu as pltpu
from jax.experimental.pallas import tpu_sc as plsc

BB = 2
DD = 256
NH = 8
NL = 4
NPT = 4
DH = 32
DFF = 1024
SHP = ((100, 100), (50, 50), (25, 25), (13, 13))
QQ = sum(h * w for h, w in SHP)
TQ = 512
QP = 13824
NQT = QP // TQ
NW = 32
ITEMS = BB * NH * QP
IPW = ITEMS // NW
CH = 8
ROUNDS = IPW // CH
RVAL = BB * NH * QP
RHALF = RVAL // 2
START1 = SHP[0][0] * SHP[0][1]
NRES = QQ - START1
NP0 = 2 * NPT
NIR = 3 * 16
NWT = 64

_STARTS = []
_s = 0
for _h, _w in SHP:
    _STARTS.append(_s)
    _s += _h * _w
_LSTART = [0, SHP[1][0] * SHP[1][1], SHP[1][0] * SHP[1][1] + SHP[2][0] * SHP[2][1]]

_COLS = np.arange(NH * NL * NPT)
_HCOL = _COLS // (NL * NPT)
_LCOL = (_COLS % (NL * NPT)) // NPT
_PCOL = _COLS % NPT
_CW = np.array([SHP[l][1] for l in _LCOL], np.float32)[None, :]
_CHT = np.array([SHP[l][0] for l in _LCOL], np.float32)[None, :]
_CLS = np.array([0 if l == 0 else _LSTART[l - 1] for l in _LCOL],
                np.float32)[None, :]
_CHQ2 = (np.arange(NH * NP0) // NP0 * (QP // 2)).astype(np.float32)[None, :]
_EXP = (np.arange(NL)[:, None] == _LCOL[None, :]).astype(np.float32)
_GSUM = (_COLS[:, None] // 16 == _COLS[None, :] // 16).astype(np.float32)


def _perm(out_cols, n_out, mask=None):
    m = np.zeros((NH * NL * NPT, n_out), np.float32)
    for c in range(NH * NL * NPT):
        if mask is None or mask[c]:
            m[c, out_cols[c]] = 1.0
    return m


_P0 = [_perm(_HCOL * NP0 + cy * NPT + _PCOL, NH * NP0, _LCOL == 0)
       for cy in range(2)]
_PR = [_perm(_HCOL * NIR + (_LCOL - 1) * 16 + cidx * NPT + _PCOL, NH * NIR,
             _LCOL >= 1) for cidx in range(4)]
_PWA = [_perm(_HCOL * NWT + cy * NPT + _PCOL, NH * NWT, _LCOL == 0)
        for cy in range(2)]
_PWB = [_perm(_HCOL * NWT + NP0 + cy * NPT + _PCOL, NH * NWT, _LCOL == 0)
        for cy in range(2)]
_PWR = [_perm(_HCOL * NWT + 16 * _LCOL + cidx * NPT + _PCOL, NH * NWT,
              _LCOL >= 1) for cidx in range(4)]


def _ln(x, g, b):
    m = x.mean(-1, keepdims=True)
    v = ((x - m) ** 2).mean(-1, keepdims=True)
    return (x - m) / jnp.sqrt(v + 1e-5) * g + b


def _hp(a, b):
    return jnp.dot(a, b, precision=lax.Precision.HIGHEST)


def _pex(x, p):
    xh = jnp.floor(x * (1.0 / 256.0)) * 256.0
    return (xh @ p) + ((x - xh) @ p)


def _pw(x, p):
    xh = x.astype(jnp.bfloat16).astype(jnp.float32)
    return (xh @ p) + ((x - xh) @ p)


def _stage_a(src_ref, rpx_ref, rpy_ref, wox_ref, box_ref, woy_ref, boy_ref,
             wa_ref, ba_ref, wv_ref, bv_ref, exp_ref, cw_ref, ch_ref, cls_ref,
             chq_ref, gsum_ref, p0a_ref, p0b_ref, pr0_ref, pr1_ref, pr2_ref,
             pr3_ref, pwa0_ref, pwa1_ref, pwb0_ref, pwb1_ref, pwr0_ref,
             pwr1_ref, pwr2_ref, pwr3_ref, val_ref, idx0_ref, idxr_ref,
             w_ref):
    b = pl.program_id(0)
    qt = pl.program_id(1)
    s = src_ref[0]
    offx = s @ wox_ref[...] + box_ref[...]
    offy = s @ woy_ref[...] + boy_ref[...]
    logits = s @ wa_ref[...] + ba_ref[...]
    val = s @ wv_ref[...] + bv_ref[...]

    rpx = _hp(rpx_ref[0], exp_ref[...])
    rpy = _hp(rpy_ref[0], exp_ref[...])
    cw = cw_ref[...]
    ch = ch_ref[...]

    e = jnp.exp(logits - logits.max(axis=-1, keepdims=True))
    a = e / _pw(e, gsum_ref[...])

    x = rpx * cw + offx - 0.5
    y = rpy * ch + offy - 0.5
    x0 = jnp.floor(x)
    y0 = jnp.floor(y)
    wx1 = x - x0
    wx0 = 1.0 - wx1
    wy1 = y - y0
    wy0 = 1.0 - wy1
    ax0 = jnp.where((x0 >= 0) & (x0 <= cw - 1), wx0, 0.0)
    ax1 = jnp.where((x0 + 1 >= 0) & (x0 + 1 <= cw - 1), wx1, 0.0)
    ay0 = jnp.where((y0 >= 0) & (y0 <= ch - 1), wy0, 0.0) * a
    ay1 = jnp.where((y0 + 1 >= 0) & (y0 + 1 <= ch - 1), wy1, 0.0) * a
    wax = jnp.where(x0 <= -1.0, ax1, ax0)
    wbx = jnp.where(x0 >= 0.0, ax1, 0.0)
    xi0 = jnp.clip(x0, 0.0, cw - 1)
    xi1 = jnp.clip(x0 + 1, 0.0, cw - 1)
    yi0 = jnp.clip(y0, 0.0, ch - 1)
    yi1 = jnp.clip(y0 + 1, 0.0, ch - 1)

    gs0 = yi0 * cw + xi0
    gs1 = yi1 * cw + xi0
    h0 = jnp.floor(gs0 * 0.5)
    h1 = jnp.floor(gs1 * 0.5)
    q0 = gs0 - 2.0 * h0
    q1 = gs1 - 2.0 * h1
    bb = (b * (NH * QP // 2)).astype(jnp.float32)

    cls = cls_ref[...]
    lr00 = cls + yi0 * cw + xi0
    lr01 = cls + yi0 * cw + xi1
    lr10 = cls + yi1 * cw + xi0
    lr11 = cls + yi1 * cw + xi1
    w00 = ay0 * ax0
    w01 = ay0 * ax1
    w10 = ay1 * ax0
    w11 = ay1 * ax1

    idx0_f = (_pex(h0, p0a_ref[...]) + _pex(h1, p0b_ref[...])
              + ((q0 @ p0a_ref[...]) + (q1 @ p0b_ref[...])) * RHALF
              + chq_ref[...] + bb)
    idxr_f = (_pex(lr00, pr0_ref[...]) + _pex(lr01, pr1_ref[...])
              + _pex(lr10, pr2_ref[...]) + _pex(lr11, pr3_ref[...])
              ) * float(DH)
    w_all = (_pw(ay0 * wax, pwa0_ref[...]) + _pw(ay1 * wax, pwa1_ref[...])
             + _pw(ay0 * wbx, pwb0_ref[...]) + _pw(ay1 * wbx, pwb1_ref[...])
             + _pw(w00, pwr0_ref[...]) + _pw(w01, pwr1_ref[...])
             + _pw(w10, pwr2_ref[...]) + _pw(w11, pwr3_ref[...]))

    qmask = (qt * TQ + lax.broadcasted_iota(jnp.int32, (TQ, 1), 0)) < QQ
    idx0_i = jnp.where(qmask, idx0_f + 0.5, 0.0).astype(jnp.int32)
    idxr_i = jnp.where(qmask, idxr_f + 0.5, 0.0).astype(jnp.int32)
    w_all = jnp.where(qmask, w_all, 0.0)

    for h in range(NH):
        idx0_ref[0, h] = idx0_i[:, h * NP0:(h + 1) * NP0]
        idxr_ref[0, h] = idxr_i[:, h * NIR:(h + 1) * NIR]
        w_ref[0, h] = w_all[:, h * NWT:(h + 1) * NWT]
        val_ref[0, h] = val[:, h * DH:(h + 1) * DH]


_GDN = lax.GatherDimensionNumbers(offset_dims=(), collapsed_slice_dims=(0,),
                                  start_index_map=(0,))


def _bcast_lane(v, zeros16, t):
    return lax.gather(v, (zeros16 + t).reshape(16, 1), _GDN, (1,),
                      mode=lax.GatherScatterMode.PROMISE_IN_BOUNDS)


def _sc_body(pair_hbm, vf_hbm, idx0_hbm, idxr_hbm, w_hbm, out_hbm,
             val_res, idx0_v, idxr_v, w_v, rows_v, out_v,
             sem_iw0, sem_iw1, sem_iw2, sem_iw3, sem_g0, sem_g1,
             sem_o0, sem_o1):
    cid = lax.axis_index("c")
    sid = lax.axis_index("s")
    wid = sid * 2 + cid
    base_item = wid * IPW
    iota16 = lax.broadcasted_iota(jnp.int32, (16,), 0)
    zeros16 = iota16 - iota16
    sem_iw = [sem_iw0, sem_iw1, sem_iw2, sem_iw3]
    sem_g = [sem_g0, sem_g1]
    sem_o = [sem_o0, sem_o1]
    last = ROUNDS - 1

    row0 = (wid >> 1) * QP
    pltpu.sync_copy(vf_hbm.at[pl.ds((row0 + START1) * DH, NRES * DH)], val_res)

    def clamp(g):
        return jnp.minimum(g, last) if not isinstance(g, int) else min(g, last)

    def fire_iw(g, buf):
        it0 = base_item + clamp(g) * CH
        pltpu.async_copy(idx0_hbm.at[pl.ds(it0 * NP0, CH * NP0)],
                         idx0_v.at[buf], sem_iw[buf])
        pltpu.async_copy(idxr_hbm.at[pl.ds(it0 * NIR, CH * NIR)],
                         idxr_v.at[buf], sem_iw[buf])
        pltpu.async_copy(w_hbm.at[pl.ds(it0 * NWT, CH * NWT)],
                         w_v.at[buf], sem_iw[buf])

    def wait_iw(buf):
        pltpu.make_async_copy(idx0_hbm.at[pl.ds(0, CH * NP0)],
                              idx0_v.at[buf], sem_iw[buf]).wait()
        pltpu.make_async_copy(idxr_hbm.at[pl.ds(0, CH * NIR)],
                              idxr_v.at[buf], sem_iw[buf]).wait()
        pltpu.make_async_copy(w_hbm.at[pl.ds(0, CH * NWT)],
                              w_v.at[buf], sem_iw[buf]).wait()

    def fire_gathers(ib, rb):
        pltpu.async_copy(pair_hbm.at[idx0_v.at[ib]], rows_v.at[rb], sem_g[rb])

    def wait_gathers(ib, rb):
        pltpu.make_async_copy(pair_hbm.at[idx0_v.at[ib]], rows_v.at[rb],
                              sem_g[rb]).wait()

    def wait_out(rb):
        pltpu.make_async_copy(out_v.at[rb],
                              out_hbm.at[pl.ds(0, CH * DH)], sem_o[rb]).wait()

    def compute(g, rb, wb):
        it0 = base_item + g * CH
        wait_out(rb)

        def item_body(i, c2):
            accs = [jnp.zeros((16,), jnp.float32) for _ in range(8)]
            wv0 = w_v[wb, pl.ds(i * NWT, 16)]
            for u in range(NP0):
                wa = _bcast_lane(wv0, zeros16, u)
                wbb = _bcast_lane(wv0, zeros16, u + 8)
                row = i * NP0 + u
                a0 = 4 * (u % 2)
                accs[a0] = accs[a0] + wa * rows_v[rb, row, pl.ds(0, 16)]
                accs[a0 + 1] = accs[a0 + 1] + wa * rows_v[rb, row,
                                                          pl.ds(16, 16)]
                accs[a0 + 2] = accs[a0 + 2] + wbb * rows_v[rb, row,
                                                           pl.ds(32, 16)]
                accs[a0 + 3] = accs[a0 + 3] + wbb * rows_v[rb, row,
                                                           pl.ds(48, 16)]
            for l in range(1, NL):
                iv = idxr_v[wb, pl.ds(i * NIR + (l - 1) * 16, 16)]
                wv = w_v[wb, pl.ds(i * NWT + l * 16, 16)]
                for t in range(16):
                    wo = _bcast_lane(iv, zeros16, t)
                    wbb = _bcast_lane(wv, zeros16, t)
                    d0 = plsc.load_gather(val_res, [wo + iota16])
                    d1 = plsc.load_gather(val_res, [wo + (iota16 + 16)])
                    a0 = 2 * (t % 4)
                    accs[a0] = accs[a0] + wbb * d0
                    accs[a0 + 1] = accs[a0 + 1] + wbb * d1
            out_v[rb, pl.ds(i * DH, 16)] = ((accs[0] + accs[2])
                                            + (accs[4] + accs[6]))
            out_v[rb, pl.ds(i * DH + 16, 16)] = ((accs[1] + accs[3])
                                                 + (accs[5] + accs[7]))
            return c2

        lax.fori_loop(0, CH, item_body, 0)
        pltpu.async_copy(out_v.at[rb],
                         out_hbm.at[pl.ds(it0 * DH, CH * DH)], sem_o[rb])

    pltpu.async_copy(w_hbm.at[pl.ds(0, CH * DH)], out_v.at[0], sem_o[0])
    pltpu.async_copy(w_hbm.at[pl.ds(0, CH * DH)], out_v.at[1], sem_o[1])
    fire_iw(0, 0)
    fire_iw(1, 1)
    fire_iw(2, 2)
    wait_iw(0)
    fire_gathers(0, 0)

    def super_round(k, carry):
        for d in range(4):
            g = 4 * k + d
            wait_iw((d + 1) % 4)
            fire_gathers((d + 1) % 4, (d + 1) % 2)
            wait_gathers(d % 4, d % 2)
            compute(g, d % 2, d % 4)
            fire_iw(g + 3, (d + 3) % 4)
        return carry

    lax.fori_loop(0, ROUNDS // 4, super_round, 0)
    wait_iw(1)
    wait_iw(2)
    wait_gathers(0, 0)
    wait_out(0)
    wait_out(1)


@functools.lru_cache(maxsize=1)
def _make_sc_sample():
    return pl.kernel(
        _sc_body,
        out_type=jax.ShapeDtypeStruct((ITEMS * DH,), jnp.float32),
        mesh=plsc.VectorSubcoreMesh(core_axis_name="c", subcore_axis_name="s"),
        scratch_types=[
            pltpu.VMEM((NRES * DH,), jnp.float32),
            pltpu.VMEM((4, CH * NP0), jnp.int32),
            pltpu.VMEM((4, CH * NIR), jnp.int32),
            pltpu.VMEM((4, CH * NWT), jnp.float32),
            pltpu.VMEM((2, CH * NP0, 2 * DH), jnp.float32),
            pltpu.VMEM((2, CH * DH), jnp.float32),
            pltpu.SemaphoreType.DMA,
            pltpu.SemaphoreType.DMA,
            pltpu.SemaphoreType.DMA,
            pltpu.SemaphoreType.DMA,
            pltpu.SemaphoreType.DMA,
            pltpu.SemaphoreType.DMA,
            pltpu.SemaphoreType.DMA,
            pltpu.SemaphoreType.DMA,
        ],
        compiler_params=pltpu.CompilerParams(use_tc_tiling_on_sc=False,
                                             needs_layout_passes=False),
    )


def _stage_b(smp_ref, src_ref, wo_ref, bo_ref, w1_ref, b1_ref, w2_ref, b2_ref,
             g1_ref, be1_ref, g2_ref, be2_ref, out_ref):
    s_cat = jnp.concatenate([smp_ref[0, h] for h in range(NH)], axis=1)
    x = s_cat @ wo_ref[...] + bo_ref[...] + src_ref[0]
    x = _ln(x, g1_ref[...], be1_ref[...])
    hid = jnp.maximum(x @ w1_ref[...] + b1_ref[...], 0.0)
    y = x + hid @ w2_ref[...] + b2_ref[...]
    out_ref[0] = _ln(y, g2_ref[...], be2_ref[...])


def _full(shape):
    nd = len(shape)
    return pl.BlockSpec(shape, lambda b, q: (0,) * nd)


def kernel(src, pos, reference_points, W_off, b_off, W_attn, b_attn, Wv, bv,
           Wo, bo, W1, b1, W2, b2, g1, be1, g2, be2, spatial_shapes):
    f32 = jnp.float32
    srcp = jnp.pad(src, ((0, 0), (0, QP - QQ), (0, 0)))
    rpx = jnp.pad(reference_points[..., 0], ((0, 0), (0, QP - QQ), (0, 0)))
    rpy = jnp.pad(reference_points[..., 1], ((0, 0), (0, QP - QQ), (0, 0)))
    wor = W_off.reshape(DD, NH, NL, NPT, 2)
    wox = wor[..., 0].reshape(DD, NH * NL * NPT)
    woy = wor[..., 1].reshape(DD, NH * NL * NPT)
    bor = b_off.reshape(NH, NL, NPT, 2)
    box = bor[..., 0].reshape(1, NH * NL * NPT)
    boy = bor[..., 1].reshape(1, NH * NL * NPT)
    ba = b_attn.reshape(1, -1)
    bv2 = bv.reshape(1, -1)

    nc = NH * NL * NPT
    val, idx0, idxr, wg = pl.pallas_call(
        _stage_a,
        grid=(BB, NQT),
        in_specs=[
            pl.BlockSpec((1, TQ, DD), lambda b, q: (b, q, 0)),
            pl.BlockSpec((1, TQ, NL), lambda b, q: (b, q, 0)),
            pl.BlockSpec((1, TQ, NL), lambda b, q: (b, q, 0)),
            _full((DD, nc)),
            _full((1, nc)),
            _full((DD, nc)),
            _full((1, nc)),
            _full((DD, nc)),
            _full((1, nc)),
            _full((DD, DD)),
            _full((1, DD)),
            _full((NL, nc)),
            _full((1, nc)),
            _full((1, nc)),
            _full((1, nc)),
            _full((1, NH * NP0)),
            _full((nc, nc)),
            _full((nc, NH * NP0)),
            _full((nc, NH * NP0)),
            _full((nc, NH * NIR)),
            _full((nc, NH * NIR)),
            _full((nc, NH * NIR)),
            _full((nc, NH * NIR)),
            _full((nc, NH * NWT)),
            _full((nc, NH * NWT)),
            _full((nc, NH * NWT)),
            _full((nc, NH * NWT)),
            _full((nc, NH * NWT)),
            _full((nc, NH * NWT)),
            _full((nc, NH * NWT)),
            _full((nc, NH * NWT)),
        ],
        out_specs=[
            pl.BlockSpec((1, NH, TQ, DH), lambda b, q: (b, 0, q, 0)),
            pl.BlockSpec((1, NH, TQ, NP0), lambda b, q: (b, 0, q, 0)),
            pl.BlockSpec((1, NH, TQ, NIR), lambda b, q: (b, 0, q, 0)),
            pl.BlockSpec((1, NH, TQ, NWT), lambda b, q: (b, 0, q, 0)),
        ],
        out_shape=[
            jax.ShapeDtypeStruct((BB, NH, QP, DH), f32),
            jax.ShapeDtypeStruct((BB, NH, QP, NP0), jnp.int32),
            jax.ShapeDtypeStruct((BB, NH, QP, NIR), jnp.int32),
            jax.ShapeDtypeStruct((BB, NH, QP, NWT), f32),
        ],
    )(srcp, rpx, rpy, wox, box, woy, boy, W_attn, ba, Wv, bv2,
      jnp.asarray(_EXP), jnp.asarray(_CW), jnp.asarray(_CHT),
      jnp.asarray(_CLS), jnp.asarray(_CHQ2), jnp.asarray(_GSUM),
      jnp.asarray(_P0[0]), jnp.asarray(_P0[1]),
      jnp.asarray(_PR[0]), jnp.asarray(_PR[1]),
      jnp.asarray(_PR[2]), jnp.asarray(_PR[3]),
      jnp.asarray(_PWA[0]), jnp.asarray(_PWA[1]),
      jnp.asarray(_PWB[0]), jnp.asarray(_PWB[1]),
      jnp.asarray(_PWR[0]), jnp.asarray(_PWR[1]),
      jnp.asarray(_PWR[2]), jnp.asarray(_PWR[3]))

    vf = val.reshape(RVAL * DH)
    even = vf.reshape(RHALF, 2 * DH)
    odd = jnp.concatenate([vf[DH:], jnp.zeros((DH,), f32)]).reshape(
        RHALF, 2 * DH)
    val_pair = jnp.concatenate([even, odd], axis=0)

    smp_flat = _make_sc_sample()(val_pair, vf,
                                 idx0.reshape(ITEMS * NP0),
                                 idxr.reshape(ITEMS * NIR),
                                 wg.reshape(ITEMS * NWT))
    smp = smp_flat.reshape(BB, NH, QP, DH)

    out = pl.pallas_call(
        _stage_b,
        grid=(BB, NQT),
        in_specs=[
            pl.BlockSpec((1, NH, TQ, DH), lambda b, q: (b, 0, q, 0)),
            pl.BlockSpec((1, TQ, DD), lambda b, q: (b, q, 0)),
            _full((DD, DD)),
            _full((1, DD)),
            _full((DD, DFF)),
            _full((1, DFF)),
            _full((DFF, DD)),
            _full((1, DD)),
            _full((1, DD)),
            _full((1, DD)),
            _full((1, DD)),
            _full((1, DD)),
        ],
        out_specs=pl.BlockSpec((1, TQ, DD), lambda b, q: (b, q, 0)),
        out_shape=jax.ShapeDtypeStruct((BB, QP, DD), f32),
    )(smp, srcp, Wo, bo.reshape(1, -1), W1, b1.reshape(1, -1),
      W2, b2.reshape(1, -1), g1.reshape(1, -1), be1.reshape(1, -1),
      g2.reshape(1, -1), be2.reshape(1, -1))

    return out[:, :QQ, :]
